# baseline (speedup 1.0000x reference)
import jax
import jax.numpy as jnp
from jax.experimental import pallas as pl
from jax.experimental.pallas import tpu as pltpu

FUSION_THRESHOLD = 0.5
_BB = 2


def _fusion_body(x_ref, sw_ref, sb_ref, cw_ref, cb_ref, o_ref):
    x = x_ref[...]
    bb, s, d = x.shape
    w1 = sw_ref[0:1, :]
    w2 = sw_ref[1:2, :]
    u = jnp.sum(x * w1, axis=2, keepdims=True)
    v = jnp.sum(x * w2, axis=2, keepdims=True)
    v_next = jnp.concatenate([v[:, 1:], v[:, -1:]], axis=1)
    score = u + v_next + sb_ref[0, 0]
    row = jax.lax.broadcasted_iota(jnp.int32, (bb, s, 1), 1)
    fuse = (score >= FUSION_THRESHOLD) & (row < s - 1)
    o_ref[...] = x

    @pl.when(jnp.any(fuse))
    def _():
        xn = jnp.concatenate([x[:, 1:], x[:, -1:]], axis=1)
        fused = (
            jax.lax.dot_general(
                x, cw_ref[0:d, :], (((2,), (0,)), ((), ())),
                preferred_element_type=jnp.float32,
            )
            + jax.lax.dot_general(
                xn, cw_ref[d:, :], (((2,), (0,)), ((), ())),
                preferred_element_type=jnp.float32,
            )
            + cb_ref[0:1, :]
        )
        o_ref[...] = jnp.where(fuse, fused, x)


def kernel(frame_input, score_w, score_b, comb_w, comb_b):
    b, s, d = frame_input.shape
    bb = _BB if b % _BB == 0 else 1
    sw = score_w.reshape(2, d)
    sb = score_b.reshape(1, 1)
    cb = comb_b.reshape(1, d)
    return pl.pallas_call(
        _fusion_body,
        grid=(b // bb,),
        in_specs=[
            pl.BlockSpec((bb, s, d), lambda i: (i, 0, 0)),
            pl.BlockSpec((2, d), lambda i: (0, 0)),
            pl.BlockSpec(memory_space=pltpu.SMEM),
            pl.BlockSpec((2 * d, d), lambda i: (0, 0)),
            pl.BlockSpec((1, d), lambda i: (0, 0)),
        ],
        out_specs=pl.BlockSpec((bb, s, d), lambda i: (i, 0, 0)),
        out_shape=jax.ShapeDtypeStruct((b, s, d), frame_input.dtype),
        compiler_params=pltpu.CompilerParams(dimension_semantics=("parallel",)),
    )(frame_input, sw, sb, comb_w, cb)


# speedup vs baseline: 1.0210x; 1.0210x over previous
import jax
import jax.numpy as jnp
from jax.experimental import pallas as pl
from jax.experimental.pallas import tpu as pltpu

FUSION_THRESHOLD = 0.5
_BB = 2


def _fusion_body(x_ref, sw_ref, sb_ref, cw_ref, cb_ref, o_ref):
    x = x_ref[...]
    bb, s, d = x.shape
    uv = jax.lax.dot_general(
        x, sw_ref[...], (((2,), (0,)), ((), ())),
        preferred_element_type=jnp.float32,
    )
    u = uv[:, :, 0:1]
    v = uv[:, :, 1:2]
    v_next = jnp.concatenate([v[:, 1:], v[:, -1:]], axis=1)
    score = u + v_next + sb_ref[0, 0]
    row = jax.lax.broadcasted_iota(jnp.int32, (bb, s, 1), 1)
    fuse = (score >= FUSION_THRESHOLD) & (row < s - 1)
    o_ref[...] = x

    @pl.when(jnp.any(fuse))
    def _():
        xn = jnp.concatenate([x[:, 1:], x[:, -1:]], axis=1)
        fused = (
            jax.lax.dot_general(
                x, cw_ref[0:d, :], (((2,), (0,)), ((), ())),
                preferred_element_type=jnp.float32,
            )
            + jax.lax.dot_general(
                xn, cw_ref[d:, :], (((2,), (0,)), ((), ())),
                preferred_element_type=jnp.float32,
            )
            + cb_ref[0:1, :]
        )
        o_ref[...] = jnp.where(fuse, fused, x)


def kernel(frame_input, score_w, score_b, comb_w, comb_b):
    b, s, d = frame_input.shape
    bb = _BB if b % _BB == 0 else 1
    sw = score_w.reshape(2, d).T
    sb = score_b.reshape(1, 1)
    cb = comb_b.reshape(1, d)
    return pl.pallas_call(
        _fusion_body,
        grid=(b // bb,),
        in_specs=[
            pl.BlockSpec((bb, s, d), lambda i: (i, 0, 0)),
            pl.BlockSpec((d, 2), lambda i: (0, 0)),
            pl.BlockSpec(memory_space=pltpu.SMEM),
            pl.BlockSpec((2 * d, d), lambda i: (0, 0)),
            pl.BlockSpec((1, d), lambda i: (0, 0)),
        ],
        out_specs=pl.BlockSpec((bb, s, d), lambda i: (i, 0, 0)),
        out_shape=jax.ShapeDtypeStruct((b, s, d), frame_input.dtype),
        compiler_params=pltpu.CompilerParams(dimension_semantics=("parallel",)),
    )(frame_input, sw, sb, comb_w, cb)
